# baseline (speedup 1.0000x reference)
import functools

import jax
import jax.numpy as jnp
from jax import lax
from jax.experimental import pallas as pl
from jax.experimental.pallas import tpu as pltpu
from jax.experimental.pallas import tpu_sc as plsc


@functools.lru_cache(maxsize=None)
def _make_gather(V, D, B):
    info = plsc.get_sparse_core_info()
    NC, NS = info.num_cores, info.num_subcores
    NW = NC * NS
    assert B % NW == 0 and (B // NW) % 8 == 0
    b_per_w = B // NW
    mesh = plsc.VectorSubcoreMesh(core_axis_name="c", subcore_axis_name="s")

    @functools.partial(
        pl.kernel,
        mesh=mesh,
        out_type=jax.ShapeDtypeStruct((B, D), jnp.float32),
        scratch_types=[
            pltpu.VMEM((b_per_w,), jnp.int32),
            pltpu.VMEM((b_per_w, D), jnp.float32),
            pltpu.SemaphoreType.DMA,
        ],
        compiler_params=pltpu.CompilerParams(use_tc_tiling_on_sc=False),
    )
    def gather_kernel(table_hbm, idx_hbm, out_hbm, idx_v, rows_v, sem):
        wid = lax.axis_index("s") * NC + lax.axis_index("c")
        base = wid * b_per_w
        pltpu.sync_copy(idx_hbm.at[pl.ds(base, b_per_w)], idx_v)
        pltpu.async_copy(table_hbm.at[idx_v], rows_v, sem).wait()
        pltpu.sync_copy(rows_v, out_hbm.at[pl.ds(base, b_per_w)])

    return gather_kernel


def kernel(target_user_weight, user_ids):
    V, D = target_user_weight.shape
    (B,) = user_ids.shape
    return _make_gather(V, D, B)(target_user_weight, user_ids)


# speedup vs baseline: 1.8212x; 1.8212x over previous
import functools

import jax
import jax.numpy as jnp
from jax import lax
from jax.experimental import pallas as pl
from jax.experimental.pallas import tpu as pltpu
from jax.experimental.pallas import tpu_sc as plsc


@functools.lru_cache(maxsize=None)
def _make_sweep(V, D, B):
    info = plsc.get_sparse_core_info()
    NC, NS = info.num_cores, info.num_subcores
    NW = NC * NS
    assert D == NW, "one embedding dim per vector subcore"
    assert B % 128 == 0
    VA = (V // 128) * 128
    TAIL = V - VA
    CH = min(VA, 44928)
    NCH = -(-VA // CH)
    LASTC = VA - (NCH - 1) * CH
    mesh = plsc.VectorSubcoreMesh(core_axis_name="c", subcore_axis_name="s")

    @functools.partial(
        pl.kernel,
        mesh=mesh,
        out_type=jax.ShapeDtypeStruct((D, B), jnp.float32),
        scratch_types=[
            pltpu.VMEM((B,), jnp.int32),
            pltpu.VMEM((B,), jnp.float32),
            pltpu.VMEM((D, max(TAIL, 1)), jnp.float32),
            pltpu.VMEM((CH,), jnp.float32),
            pltpu.VMEM((CH,), jnp.float32),
            pltpu.SemaphoreType.DMA,
            pltpu.SemaphoreType.DMA,
        ],
        compiler_params=pltpu.CompilerParams(needs_layout_passes=False),
    )
    def sweep_kernel(table_hbm, tail_hbm, idx_hbm, out_hbm,
                     idx_v, row_v, tail_v, buf0, buf1, sem0, sem1):
        w = lax.axis_index("s") * NC + lax.axis_index("c")
        pltpu.sync_copy(idx_hbm, idx_v)
        if TAIL:
            pltpu.sync_copy(tail_hbm, tail_v)
        bufs, sems = [buf0, buf1], [sem0, sem1]
        copies = {}
        copies[0] = pltpu.async_copy(
            table_hbm.at[w].at[pl.ds(0, CH)], buf0, sem0)
        iota16 = lax.iota(jnp.int32, 16)
        wv = iota16 * 0 + w
        for k in range(NCH):
            size_k = CH if k < NCH - 1 else LASTC
            copies[k].wait()
            if k + 1 < NCH:
                nsize = CH if k + 1 < NCH - 1 else LASTC
                copies[k + 1] = pltpu.async_copy(
                    table_hbm.at[w].at[pl.ds((k + 1) * CH, nsize)],
                    bufs[(k + 1) % 2].at[pl.ds(0, nsize)],
                    sems[(k + 1) % 2],
                )
            buf = bufs[k % 2]
            lo = k * CH
            hi = lo + size_k

            def scan_body(j, _, buf=buf, lo=lo, hi=hi):
                for u in range(8):
                    pos = j * 128 + u * 16
                    iv = idx_v[pl.ds(pos, 16)]
                    m = (iv >= lo) & (iv < hi)
                    local = jnp.where(m, iv - lo, 0)
                    val = plsc.load_gather(buf, [local], mask=m)
                    plsc.store_scatter(row_v, [iota16 + pos], val, mask=m)
                return ()

            lax.fori_loop(0, B // 128, scan_body, ())

        if TAIL:
            def tail_body(j, _):
                for u in range(8):
                    pos = j * 128 + u * 16
                    iv = idx_v[pl.ds(pos, 16)]
                    m = iv >= VA
                    local = jnp.where(m, iv - VA, 0)
                    val = plsc.load_gather(tail_v, [wv, local], mask=m)
                    plsc.store_scatter(row_v, [iota16 + pos], val, mask=m)
                return ()

            lax.fori_loop(0, B // 128, tail_body, ())
        pltpu.sync_copy(row_v, out_hbm.at[w])

    return sweep_kernel, VA, TAIL


def kernel(target_user_weight, user_ids):
    V, D = target_user_weight.shape
    (B,) = user_ids.shape
    sweep, VA, TAIL = _make_sweep(V, D, B)
    tail = target_user_weight[VA:, :].T if TAIL else (
        jnp.zeros((D, 1), jnp.float32))
    out_t = sweep(target_user_weight.T, tail, user_ids)
    return out_t.T


# speedup vs baseline: 1.8220x; 1.0004x over previous
import functools

import jax
import jax.numpy as jnp
from jax import lax
from jax.experimental import pallas as pl
from jax.experimental.pallas import tpu as pltpu
from jax.experimental.pallas import tpu_sc as plsc


@functools.lru_cache(maxsize=None)
def _make_sweep(V, D, B):
    info = plsc.get_sparse_core_info()
    NC, NS = info.num_cores, info.num_subcores
    NW = NC * NS
    assert D == NW, "one embedding dim per vector subcore"
    assert B % 128 == 0
    VA = (V // 128) * 128
    TAIL = V - VA
    CH = min(VA, 44928)
    NCH = -(-VA // CH)
    LASTC = VA - (NCH - 1) * CH
    mesh = plsc.VectorSubcoreMesh(core_axis_name="c", subcore_axis_name="s")

    @functools.partial(
        pl.kernel,
        mesh=mesh,
        out_type=jax.ShapeDtypeStruct((D, B), jnp.float32),
        scratch_types=[
            pltpu.VMEM((B,), jnp.int32),
            pltpu.VMEM((B,), jnp.float32),
            pltpu.VMEM((D, max(TAIL, 1)), jnp.float32),
            pltpu.VMEM((CH,), jnp.float32),
            pltpu.VMEM((CH,), jnp.float32),
            pltpu.SemaphoreType.DMA,
            pltpu.SemaphoreType.DMA,
        ],
        compiler_params=pltpu.CompilerParams(needs_layout_passes=False),
    )
    def sweep_kernel(table_hbm, tail_hbm, idx_hbm, out_hbm,
                     idx_v, row_v, tail_v, buf0, buf1, sem0, sem1):
        w = lax.axis_index("s") * NC + lax.axis_index("c")
        bufs, sems = [buf0, buf1], [sem0, sem1]
        copies = {}
        copies[0] = pltpu.async_copy(
            table_hbm.at[w].at[pl.ds(0, CH)], buf0, sem0)
        idx_copy = pltpu.async_copy(idx_hbm, idx_v, sem1)
        if TAIL:
            pltpu.sync_copy(tail_hbm, tail_v)
        idx_copy.wait()
        iota16 = lax.iota(jnp.int32, 16)
        wv = iota16 * 0 + w
        UNR = 16
        for k in range(NCH):
            size_k = CH if k < NCH - 1 else LASTC
            copies[k].wait()
            if k + 1 < NCH:
                nsize = CH if k + 1 < NCH - 1 else LASTC
                copies[k + 1] = pltpu.async_copy(
                    table_hbm.at[w].at[pl.ds((k + 1) * CH, nsize)],
                    bufs[(k + 1) % 2].at[pl.ds(0, nsize)],
                    sems[(k + 1) % 2],
                )
            buf = bufs[k % 2]
            lo = k * CH
            usize = jnp.uint32(size_k)

            def scan_body(j, _, buf=buf, lo=lo, usize=usize):
                for u in range(UNR):
                    pos = j * (16 * UNR) + u * 16
                    iv = idx_v[pl.ds(pos, 16)]
                    local = iv - lo
                    m = local.astype(jnp.uint32) < usize
                    val = plsc.load_gather(buf, [local], mask=m)
                    plsc.store_scatter(row_v, [iota16 + pos], val, mask=m)
                return ()

            lax.fori_loop(0, B // (16 * UNR), scan_body, ())

        if TAIL:
            utail = jnp.uint32(TAIL)

            def tail_body(j, _):
                for u in range(UNR):
                    pos = j * (16 * UNR) + u * 16
                    iv = idx_v[pl.ds(pos, 16)]
                    local = iv - VA
                    m = local.astype(jnp.uint32) < utail
                    val = plsc.load_gather(tail_v, [wv, local], mask=m)
                    plsc.store_scatter(row_v, [iota16 + pos], val, mask=m)
                return ()

            lax.fori_loop(0, B // (16 * UNR), tail_body, ())
        pltpu.sync_copy(row_v, out_hbm.at[w])

    return sweep_kernel, VA, TAIL


def kernel(target_user_weight, user_ids):
    V, D = target_user_weight.shape
    (B,) = user_ids.shape
    sweep, VA, TAIL = _make_sweep(V, D, B)
    tail = target_user_weight[VA:, :].T if TAIL else (
        jnp.zeros((D, 1), jnp.float32))
    out_t = sweep(target_user_weight.T, tail, user_ids)
    return out_t.T


# speedup vs baseline: 3.4730x; 1.9061x over previous
import functools

import jax
import jax.numpy as jnp
from jax import lax
from jax.experimental import pallas as pl
from jax.experimental.pallas import tpu as pltpu
from jax.experimental.pallas import tpu_sc as plsc


@functools.lru_cache(maxsize=None)
def _make_sweep(V, D, B):
    info = plsc.get_sparse_core_info()
    NC, NS = info.num_cores, info.num_subcores
    NW = NC * NS
    assert D == NW, "one embedding dim per vector subcore"
    assert B % 128 == 0
    VA = (V // 128) * 128
    TAIL = V - VA
    CH = min(VA, 44928)
    NCH = -(-VA // CH)
    LASTC = VA - (NCH - 1) * CH
    mesh = plsc.VectorSubcoreMesh(core_axis_name="c", subcore_axis_name="s")

    @functools.partial(
        pl.kernel,
        mesh=mesh,
        out_type=jax.ShapeDtypeStruct((D, B), jnp.float32),
        scratch_types=[
            pltpu.VMEM((B,), jnp.int32),
            pltpu.VMEM((B,), jnp.float32),
            pltpu.VMEM((D, max(TAIL, 1)), jnp.float32),
            pltpu.VMEM((CH,), jnp.float32),
            pltpu.VMEM((CH,), jnp.float32),
            pltpu.SemaphoreType.DMA,
            pltpu.SemaphoreType.DMA,
        ],
        compiler_params=pltpu.CompilerParams(needs_layout_passes=False),
    )
    def sweep_kernel(table_hbm, tail_hbm, idx_hbm, out_hbm,
                     idx_v, row_v, tail_v, buf0, buf1, sem0, sem1):
        w = lax.axis_index("s") * NC + lax.axis_index("c")
        bufs, sems = [buf0, buf1], [sem0, sem1]
        copies = {}
        copies[0] = pltpu.async_copy(
            table_hbm.at[w].at[pl.ds(0, CH)], buf0, sem0)
        idx_copy = pltpu.async_copy(idx_hbm, idx_v, sem1)
        if TAIL:
            pltpu.sync_copy(tail_hbm, tail_v)
        idx_copy.wait()
        iota16 = lax.iota(jnp.int32, 16)
        wv = iota16 * 0 + w
        UNR = 16
        for k in range(NCH):
            size_k = CH if k < NCH - 1 else LASTC
            copies[k].wait()
            if k + 1 < NCH:
                nsize = CH if k + 1 < NCH - 1 else LASTC
                copies[k + 1] = pltpu.async_copy(
                    table_hbm.at[w].at[pl.ds((k + 1) * CH, nsize)],
                    bufs[(k + 1) % 2].at[pl.ds(0, nsize)],
                    sems[(k + 1) % 2],
                )
            buf = bufs[k % 2]
            lo = k * CH
            usize = jnp.uint32(size_k)

            def scan_body(j, _, buf=buf, lo=lo, usize=usize):
                for u in range(UNR):
                    pos = j * (16 * UNR) + u * 16
                    iv = idx_v[pl.ds(pos, 16)]
                    local = iv - lo
                    m = local.astype(jnp.uint32) < usize
                    val = plsc.load_gather(buf, [local], mask=m)
                    old = row_v[pl.ds(pos, 16)]
                    row_v[pl.ds(pos, 16)] = jnp.where(m, val, old)
                return ()

            lax.fori_loop(0, B // (16 * UNR), scan_body, ())

        if TAIL:
            utail = jnp.uint32(TAIL)

            def tail_body(j, _):
                for u in range(UNR):
                    pos = j * (16 * UNR) + u * 16
                    iv = idx_v[pl.ds(pos, 16)]
                    local = iv - VA
                    m = local.astype(jnp.uint32) < utail
                    val = plsc.load_gather(tail_v, [wv, local], mask=m)
                    old = row_v[pl.ds(pos, 16)]
                    row_v[pl.ds(pos, 16)] = jnp.where(m, val, old)
                return ()

            lax.fori_loop(0, B // (16 * UNR), tail_body, ())
        pltpu.sync_copy(row_v, out_hbm.at[w])

    return sweep_kernel, VA, TAIL


def kernel(target_user_weight, user_ids):
    V, D = target_user_weight.shape
    (B,) = user_ids.shape
    sweep, VA, TAIL = _make_sweep(V, D, B)
    tail = target_user_weight[VA:, :].T if TAIL else (
        jnp.zeros((D, 1), jnp.float32))
    out_t = sweep(target_user_weight.T, tail, user_ids)
    return out_t.T
